# baseline (speedup 1.0000x reference)
import functools

import jax
import jax.numpy as jnp
from jax import lax
from jax.experimental import pallas as pl
from jax.experimental.pallas import tpu as pltpu
from jax.experimental.pallas import tpu_sc as plsc

EMBED = 64
CHUNK = 512
TBLOCK = 512
VOCAB_PAD = 1000448


def _pack_block(tt_ref, p_ref):
    a = tt_ref[...]
    r = lax.broadcasted_iota(jnp.int32, (EMBED, EMBED), 0)
    c = lax.broadcasted_iota(jnp.int32, (EMBED, EMBED), 1)
    ident = (r == c).astype(jnp.float32)
    y = lax.dot_general(
        a, ident, (((0,), (0,)), ((), ())),
        preferred_element_type=jnp.float32)
    half = TBLOCK // 2
    p_ref[...] = jnp.concatenate([y[:half], y[half:]], axis=1)


@functools.cache
def _pack_fn(v_pad: int):
    n_blocks = v_pad // TBLOCK
    assert n_blocks * TBLOCK == v_pad
    return pl.pallas_call(
        _pack_block,
        grid=(n_blocks,),
        in_specs=[pl.BlockSpec((EMBED, TBLOCK), lambda i: (0, i))],
        out_specs=pl.BlockSpec((TBLOCK // 2, 2 * EMBED), lambda i: (i, 0)),
        out_shape=jax.ShapeDtypeStruct((v_pad // 2, 2 * EMBED), jnp.float32),
    )


@functools.cache
def _gather_fn(n_total: int, v_pad: int):
    info = plsc.get_sparse_core_info()
    nc, ns = info.num_cores, info.num_subcores
    nw = nc * ns
    per_w = n_total // nw
    n_chunks = per_w // CHUNK
    assert per_w * nw == n_total and n_chunks * CHUNK == per_w
    assert n_chunks % 2 == 0 and n_chunks >= 6

    mesh = plsc.VectorSubcoreMesh(core_axis_name="c", subcore_axis_name="s")

    @functools.partial(
        pl.kernel,
        mesh=mesh,
        out_type=jax.ShapeDtypeStruct((n_total, EMBED), jnp.float32),
        scratch_types=[
            pltpu.VMEM((CHUNK,), jnp.int32),
            pltpu.VMEM((CHUNK,), jnp.int32),
            pltpu.VMEM((CHUNK, EMBED), jnp.float32),
            pltpu.VMEM((CHUNK, EMBED), jnp.float32),
            pltpu.SemaphoreType.DMA,
            pltpu.SemaphoreType.DMA,
            pltpu.SemaphoreType.DMA,
            pltpu.SemaphoreType.DMA,
            pltpu.SemaphoreType.DMA,
            pltpu.SemaphoreType.DMA,
        ],
        compiler_params=pltpu.CompilerParams(use_tc_tiling_on_sc=False),
    )
    def k(idx_hbm, table_hbm, out_hbm, idx0, idx1, rows0, rows1,
          si0, si1, sg0, sg1, so0, so1):
        idx_v = (idx0, idx1)
        rows_v = (rows0, rows1)
        si = (si0, si1)
        sg = (sg0, sg1)
        so = (so0, so1)
        wid = lax.axis_index("s") * nc + lax.axis_index("c")
        w_base = wid * per_w

        def start_idx(i, b):
            pltpu.async_copy(
                idx_hbm.at[pl.ds(w_base + i * CHUNK, CHUNK)], idx_v[b], si[b])

        def wait_idx(b):
            pltpu.make_async_copy(
                idx_hbm.at[pl.ds(0, CHUNK)], idx_v[b], si[b]).wait()

        def start_out(i, b):
            pltpu.async_copy(
                rows_v[b], out_hbm.at[pl.ds(w_base + i * CHUNK, CHUNK)], so[b])

        def wait_out(b):
            pltpu.make_async_copy(
                rows_v[b], out_hbm.at[pl.ds(0, CHUNK)], so[b]).wait()

        def wait_gather(b):
            pltpu.make_async_copy(
                table_hbm.at[idx_v[b]], rows_v[b], sg[b]).wait()

        def xform_idx(b):
            def tb(kk, carry2):
                t = idx_v[b][pl.ds(kk * 16, 16)]
                u = (t & ~511) | ((t & 255) << 1) | ((t >> 8) & 1)
                idx_v[b][pl.ds(kk * 16, 16)] = u
                return carry2

            lax.fori_loop(0, CHUNK // 16, tb, 0)

        def step(i, b, *, slot_wait, drain_prev, prefetch):
            if slot_wait:
                wait_out(b)
            wait_idx(b)
            xform_idx(b)
            pltpu.async_copy(table_hbm.at[idx_v[b]], rows_v[b], sg[b])
            if drain_prev:
                wait_gather(1 - b)
                start_out(i - 1, 1 - b)
            if prefetch:
                start_idx(i + 1, 1 - b)

        start_idx(0, 0)
        step(0, 0, slot_wait=False, drain_prev=False, prefetch=True)
        step(1, 1, slot_wait=False, drain_prev=True, prefetch=True)

        def outer(j, carry):
            i2 = j * 2
            step(i2, 0, slot_wait=True, drain_prev=True, prefetch=True)
            step(i2 + 1, 1, slot_wait=True, drain_prev=True, prefetch=True)
            return carry

        lax.fori_loop(1, n_chunks // 2 - 1, outer, 0)

        step(n_chunks - 2, 0, slot_wait=True, drain_prev=True, prefetch=True)
        step(n_chunks - 1, 1, slot_wait=True, drain_prev=True, prefetch=False)
        wait_gather(1)
        start_out(n_chunks - 1, 1)
        wait_out(0)
        wait_out(1)

    return k


def kernel(sentences, table):
    b, s = sentences.shape
    n = b * s
    v = table.shape[0]
    packed = _pack_fn(VOCAB_PAD)(table.T)
    table_rm = packed.reshape(VOCAB_PAD, EMBED)
    idx = sentences.T.reshape(n).astype(jnp.int32)
    out = _gather_fn(n, VOCAB_PAD)(idx, table_rm)
    return out.reshape(s, b, EMBED).transpose(1, 0, 2)


# speedup vs baseline: 2.0047x; 2.0047x over previous
import functools

import jax
import jax.numpy as jnp
from jax import lax
from jax.experimental import pallas as pl
from jax.experimental.pallas import tpu as pltpu
from jax.experimental.pallas import tpu_sc as plsc

EMBED = 64
CHUNK = 512
TBLOCK = 4096
VOCAB_PAD = 1003520


def _pack_block(tt_ref, p_ref):
    a = tt_ref[...]
    r = lax.broadcasted_iota(jnp.int32, (EMBED, EMBED), 0)
    c = lax.broadcasted_iota(jnp.int32, (EMBED, EMBED), 1)
    ident = (r == c).astype(jnp.float32)
    y = lax.dot_general(
        a, ident, (((0,), (0,)), ((), ())),
        preferred_element_type=jnp.float32)
    half = TBLOCK // 2
    p_ref[...] = jnp.concatenate([y[:half], y[half:]], axis=1)


@functools.cache
def _pack_fn(v_pad: int):
    n_blocks = v_pad // TBLOCK
    assert n_blocks * TBLOCK == v_pad
    return pl.pallas_call(
        _pack_block,
        grid=(n_blocks,),
        in_specs=[pl.BlockSpec((EMBED, TBLOCK), lambda i: (0, i))],
        out_specs=pl.BlockSpec((TBLOCK // 2, 2 * EMBED), lambda i: (i, 0)),
        out_shape=jax.ShapeDtypeStruct((v_pad // 2, 2 * EMBED), jnp.float32),
    )


@functools.cache
def _gather_fn(n_total: int, v_pad: int):
    info = plsc.get_sparse_core_info()
    nc, ns = info.num_cores, info.num_subcores
    nw = nc * ns
    per_w = n_total // nw
    n_chunks = per_w // CHUNK
    assert per_w * nw == n_total and n_chunks * CHUNK == per_w
    assert n_chunks % 2 == 0 and n_chunks >= 6

    mesh = plsc.VectorSubcoreMesh(core_axis_name="c", subcore_axis_name="s")

    @functools.partial(
        pl.kernel,
        mesh=mesh,
        out_type=jax.ShapeDtypeStruct((n_total, EMBED), jnp.float32),
        scratch_types=[
            pltpu.VMEM((CHUNK,), jnp.int32),
            pltpu.VMEM((CHUNK,), jnp.int32),
            pltpu.VMEM((CHUNK, EMBED), jnp.float32),
            pltpu.VMEM((CHUNK, EMBED), jnp.float32),
            pltpu.SemaphoreType.DMA,
            pltpu.SemaphoreType.DMA,
            pltpu.SemaphoreType.DMA,
            pltpu.SemaphoreType.DMA,
            pltpu.SemaphoreType.DMA,
            pltpu.SemaphoreType.DMA,
        ],
        compiler_params=pltpu.CompilerParams(use_tc_tiling_on_sc=False),
    )
    def k(idx_hbm, table_hbm, out_hbm, idx0, idx1, rows0, rows1,
          si0, si1, sg0, sg1, so0, so1):
        idx_v = (idx0, idx1)
        rows_v = (rows0, rows1)
        si = (si0, si1)
        sg = (sg0, sg1)
        so = (so0, so1)
        wid = lax.axis_index("s") * nc + lax.axis_index("c")
        w_base = wid * per_w

        def start_idx(i, b):
            pltpu.async_copy(
                idx_hbm.at[pl.ds(w_base + i * CHUNK, CHUNK)], idx_v[b], si[b])

        def wait_idx(b):
            pltpu.make_async_copy(
                idx_hbm.at[pl.ds(0, CHUNK)], idx_v[b], si[b]).wait()

        def start_out(i, b):
            pltpu.async_copy(
                rows_v[b], out_hbm.at[pl.ds(w_base + i * CHUNK, CHUNK)], so[b])

        def wait_out(b):
            pltpu.make_async_copy(
                rows_v[b], out_hbm.at[pl.ds(0, CHUNK)], so[b]).wait()

        def wait_gather(b):
            pltpu.make_async_copy(
                table_hbm.at[idx_v[b]], rows_v[b], sg[b]).wait()

        def xform_idx(b):
            def tb(kk, carry2):
                t = idx_v[b][pl.ds(kk * 16, 16)]
                u = (t & ~(TBLOCK - 1)) | ((t & (TBLOCK // 2 - 1)) << 1) | ((t >> 11) & 1)
                idx_v[b][pl.ds(kk * 16, 16)] = u
                return carry2

            lax.fori_loop(0, CHUNK // 16, tb, 0)

        def step(i, b, *, slot_wait, drain_prev, prefetch):
            if slot_wait:
                wait_out(b)
            wait_idx(b)
            xform_idx(b)
            pltpu.async_copy(table_hbm.at[idx_v[b]], rows_v[b], sg[b])
            if drain_prev:
                wait_gather(1 - b)
                start_out(i - 1, 1 - b)
            if prefetch:
                start_idx(i + 1, 1 - b)

        start_idx(0, 0)
        step(0, 0, slot_wait=False, drain_prev=False, prefetch=True)
        step(1, 1, slot_wait=False, drain_prev=True, prefetch=True)

        def outer(j, carry):
            i2 = j * 2
            step(i2, 0, slot_wait=True, drain_prev=True, prefetch=True)
            step(i2 + 1, 1, slot_wait=True, drain_prev=True, prefetch=True)
            return carry

        lax.fori_loop(1, n_chunks // 2 - 1, outer, 0)

        step(n_chunks - 2, 0, slot_wait=True, drain_prev=True, prefetch=True)
        step(n_chunks - 1, 1, slot_wait=True, drain_prev=True, prefetch=False)
        wait_gather(1)
        start_out(n_chunks - 1, 1)
        wait_out(0)
        wait_out(1)

    return k


def kernel(sentences, table):
    b, s = sentences.shape
    n = b * s
    v = table.shape[0]
    packed = _pack_fn(VOCAB_PAD)(table.T)
    table_rm = packed.reshape(VOCAB_PAD, EMBED)
    idx = sentences.T.reshape(n).astype(jnp.int32)
    out = _gather_fn(n, VOCAB_PAD)(idx, table_rm)
    return out.reshape(s, b, EMBED).transpose(1, 0, 2)


# speedup vs baseline: 2.1618x; 1.0784x over previous
import functools

import jax
import jax.numpy as jnp
from jax import lax
from jax.experimental import pallas as pl
from jax.experimental.pallas import tpu as pltpu
from jax.experimental.pallas import tpu_sc as plsc

EMBED = 64
CHUNK = 512
TBLOCK = 8192
HALF_SHIFT = (TBLOCK // 2).bit_length() - 1
VOCAB_PAD = 1007616


def _pack_block(tt_ref, p_ref):
    a = tt_ref[...]
    r = lax.broadcasted_iota(jnp.int32, (EMBED, EMBED), 0)
    c = lax.broadcasted_iota(jnp.int32, (EMBED, EMBED), 1)
    ident = (r == c).astype(jnp.float32)
    y = lax.dot_general(
        a, ident, (((0,), (0,)), ((), ())),
        preferred_element_type=jnp.float32)
    half = TBLOCK // 2
    p_ref[...] = jnp.concatenate([y[:half], y[half:]], axis=1)


@functools.cache
def _pack_fn(v_pad: int):
    n_blocks = v_pad // TBLOCK
    assert n_blocks * TBLOCK == v_pad
    return pl.pallas_call(
        _pack_block,
        grid=(n_blocks,),
        in_specs=[pl.BlockSpec((EMBED, TBLOCK), lambda i: (0, i))],
        out_specs=pl.BlockSpec((TBLOCK // 2, 2 * EMBED), lambda i: (i, 0)),
        out_shape=jax.ShapeDtypeStruct((v_pad // 2, 2 * EMBED), jnp.float32),
    )


@functools.cache
def _gather_fn(n_total: int, v_pad: int):
    info = plsc.get_sparse_core_info()
    nc, ns = info.num_cores, info.num_subcores
    nw = nc * ns
    per_w = n_total // nw
    n_chunks = per_w // CHUNK
    assert per_w * nw == n_total and n_chunks * CHUNK == per_w
    assert n_chunks % 2 == 0 and n_chunks >= 6

    mesh = plsc.VectorSubcoreMesh(core_axis_name="c", subcore_axis_name="s")

    @functools.partial(
        pl.kernel,
        mesh=mesh,
        out_type=jax.ShapeDtypeStruct((n_total, EMBED), jnp.float32),
        scratch_types=[
            pltpu.VMEM((CHUNK,), jnp.int32),
            pltpu.VMEM((CHUNK,), jnp.int32),
            pltpu.VMEM((CHUNK, EMBED), jnp.float32),
            pltpu.VMEM((CHUNK, EMBED), jnp.float32),
            pltpu.SemaphoreType.DMA,
            pltpu.SemaphoreType.DMA,
            pltpu.SemaphoreType.DMA,
            pltpu.SemaphoreType.DMA,
            pltpu.SemaphoreType.DMA,
            pltpu.SemaphoreType.DMA,
        ],
        compiler_params=pltpu.CompilerParams(use_tc_tiling_on_sc=False),
    )
    def k(idx_hbm, table_hbm, out_hbm, idx0, idx1, rows0, rows1,
          si0, si1, sg0, sg1, so0, so1):
        idx_v = (idx0, idx1)
        rows_v = (rows0, rows1)
        si = (si0, si1)
        sg = (sg0, sg1)
        so = (so0, so1)
        wid = lax.axis_index("s") * nc + lax.axis_index("c")
        w_base = wid * per_w

        def start_idx(i, b):
            pltpu.async_copy(
                idx_hbm.at[pl.ds(w_base + i * CHUNK, CHUNK)], idx_v[b], si[b])

        def wait_idx(b):
            pltpu.make_async_copy(
                idx_hbm.at[pl.ds(0, CHUNK)], idx_v[b], si[b]).wait()

        def start_out(i, b):
            pltpu.async_copy(
                rows_v[b], out_hbm.at[pl.ds(w_base + i * CHUNK, CHUNK)], so[b])

        def wait_out(b):
            pltpu.make_async_copy(
                rows_v[b], out_hbm.at[pl.ds(0, CHUNK)], so[b]).wait()

        def wait_gather(b):
            pltpu.make_async_copy(
                table_hbm.at[idx_v[b]], rows_v[b], sg[b]).wait()

        def xform_idx(b):
            def tb(kk, carry2):
                t = idx_v[b][pl.ds(kk * 16, 16)]
                u = (t & ~(TBLOCK - 1)) | ((t & (TBLOCK // 2 - 1)) << 1) | ((t >> HALF_SHIFT) & 1)
                idx_v[b][pl.ds(kk * 16, 16)] = u
                return carry2

            lax.fori_loop(0, CHUNK // 16, tb, 0)

        def step(i, b, *, slot_wait, drain_prev, prefetch):
            if slot_wait:
                wait_out(b)
            wait_idx(b)
            xform_idx(b)
            pltpu.async_copy(table_hbm.at[idx_v[b]], rows_v[b], sg[b])
            if drain_prev:
                wait_gather(1 - b)
                start_out(i - 1, 1 - b)
            if prefetch:
                start_idx(i + 1, 1 - b)

        start_idx(0, 0)
        step(0, 0, slot_wait=False, drain_prev=False, prefetch=True)
        step(1, 1, slot_wait=False, drain_prev=True, prefetch=True)

        def outer(j, carry):
            i2 = j * 2
            step(i2, 0, slot_wait=True, drain_prev=True, prefetch=True)
            step(i2 + 1, 1, slot_wait=True, drain_prev=True, prefetch=True)
            return carry

        lax.fori_loop(1, n_chunks // 2 - 1, outer, 0)

        step(n_chunks - 2, 0, slot_wait=True, drain_prev=True, prefetch=True)
        step(n_chunks - 1, 1, slot_wait=True, drain_prev=True, prefetch=False)
        wait_gather(1)
        start_out(n_chunks - 1, 1)
        wait_out(0)
        wait_out(1)

    return k


def kernel(sentences, table):
    b, s = sentences.shape
    n = b * s
    v = table.shape[0]
    packed = _pack_fn(VOCAB_PAD)(table.T)
    table_rm = packed.reshape(VOCAB_PAD, EMBED)
    idx = sentences.T.reshape(n).astype(jnp.int32)
    out = _gather_fn(n, VOCAB_PAD)(idx, table_rm)
    return out.reshape(s, b, EMBED).transpose(1, 0, 2)


# speedup vs baseline: 2.2409x; 1.0366x over previous
import functools

import jax
import jax.numpy as jnp
from jax import lax
from jax.experimental import pallas as pl
from jax.experimental.pallas import tpu as pltpu
from jax.experimental.pallas import tpu_sc as plsc

EMBED = 64
CHUNK = 512
TBLOCK = 16384
HALF_SHIFT = (TBLOCK // 2).bit_length() - 1
VOCAB_PAD = 1015808


def _pack_block(tt_ref, p_ref):
    a = tt_ref[...]
    r = lax.broadcasted_iota(jnp.int32, (EMBED, EMBED), 0)
    c = lax.broadcasted_iota(jnp.int32, (EMBED, EMBED), 1)
    ident = (r == c).astype(jnp.float32)
    y = lax.dot_general(
        a, ident, (((0,), (0,)), ((), ())),
        preferred_element_type=jnp.float32)
    half = TBLOCK // 2
    p_ref[...] = jnp.concatenate([y[:half], y[half:]], axis=1)


@functools.cache
def _pack_fn(v_pad: int):
    n_blocks = v_pad // TBLOCK
    assert n_blocks * TBLOCK == v_pad
    return pl.pallas_call(
        _pack_block,
        grid=(n_blocks,),
        in_specs=[pl.BlockSpec((EMBED, TBLOCK), lambda i: (0, i))],
        out_specs=pl.BlockSpec((TBLOCK // 2, 2 * EMBED), lambda i: (i, 0)),
        out_shape=jax.ShapeDtypeStruct((v_pad // 2, 2 * EMBED), jnp.float32),
    )


@functools.cache
def _gather_fn(n_total: int, v_pad: int):
    info = plsc.get_sparse_core_info()
    nc, ns = info.num_cores, info.num_subcores
    nw = nc * ns
    per_w = n_total // nw
    n_chunks = per_w // CHUNK
    assert per_w * nw == n_total and n_chunks * CHUNK == per_w
    assert n_chunks % 2 == 0 and n_chunks >= 6

    mesh = plsc.VectorSubcoreMesh(core_axis_name="c", subcore_axis_name="s")

    @functools.partial(
        pl.kernel,
        mesh=mesh,
        out_type=jax.ShapeDtypeStruct((n_total, EMBED), jnp.float32),
        scratch_types=[
            pltpu.VMEM((CHUNK,), jnp.int32),
            pltpu.VMEM((CHUNK,), jnp.int32),
            pltpu.VMEM((CHUNK, EMBED), jnp.float32),
            pltpu.VMEM((CHUNK, EMBED), jnp.float32),
            pltpu.SemaphoreType.DMA,
            pltpu.SemaphoreType.DMA,
            pltpu.SemaphoreType.DMA,
            pltpu.SemaphoreType.DMA,
            pltpu.SemaphoreType.DMA,
            pltpu.SemaphoreType.DMA,
        ],
        compiler_params=pltpu.CompilerParams(use_tc_tiling_on_sc=False),
    )
    def k(idx_hbm, table_hbm, out_hbm, idx0, idx1, rows0, rows1,
          si0, si1, sg0, sg1, so0, so1):
        idx_v = (idx0, idx1)
        rows_v = (rows0, rows1)
        si = (si0, si1)
        sg = (sg0, sg1)
        so = (so0, so1)
        wid = lax.axis_index("s") * nc + lax.axis_index("c")
        w_base = wid * per_w

        def start_idx(i, b):
            pltpu.async_copy(
                idx_hbm.at[pl.ds(w_base + i * CHUNK, CHUNK)], idx_v[b], si[b])

        def wait_idx(b):
            pltpu.make_async_copy(
                idx_hbm.at[pl.ds(0, CHUNK)], idx_v[b], si[b]).wait()

        def start_out(i, b):
            pltpu.async_copy(
                rows_v[b], out_hbm.at[pl.ds(w_base + i * CHUNK, CHUNK)], so[b])

        def wait_out(b):
            pltpu.make_async_copy(
                rows_v[b], out_hbm.at[pl.ds(0, CHUNK)], so[b]).wait()

        def wait_gather(b):
            pltpu.make_async_copy(
                table_hbm.at[idx_v[b]], rows_v[b], sg[b]).wait()

        def xform_idx(b):
            def tb(kk, carry2):
                t = idx_v[b][pl.ds(kk * 16, 16)]
                u = (t & ~(TBLOCK - 1)) | ((t & (TBLOCK // 2 - 1)) << 1) | ((t >> HALF_SHIFT) & 1)
                idx_v[b][pl.ds(kk * 16, 16)] = u
                return carry2

            lax.fori_loop(0, CHUNK // 16, tb, 0)

        def step(i, b, *, slot_wait, drain_prev, prefetch):
            if slot_wait:
                wait_out(b)
            wait_idx(b)
            xform_idx(b)
            pltpu.async_copy(table_hbm.at[idx_v[b]], rows_v[b], sg[b])
            if drain_prev:
                wait_gather(1 - b)
                start_out(i - 1, 1 - b)
            if prefetch:
                start_idx(i + 1, 1 - b)

        start_idx(0, 0)
        step(0, 0, slot_wait=False, drain_prev=False, prefetch=True)
        step(1, 1, slot_wait=False, drain_prev=True, prefetch=True)

        def outer(j, carry):
            i2 = j * 2
            step(i2, 0, slot_wait=True, drain_prev=True, prefetch=True)
            step(i2 + 1, 1, slot_wait=True, drain_prev=True, prefetch=True)
            return carry

        lax.fori_loop(1, n_chunks // 2 - 1, outer, 0)

        step(n_chunks - 2, 0, slot_wait=True, drain_prev=True, prefetch=True)
        step(n_chunks - 1, 1, slot_wait=True, drain_prev=True, prefetch=False)
        wait_gather(1)
        start_out(n_chunks - 1, 1)
        wait_out(0)
        wait_out(1)

    return k


def kernel(sentences, table):
    b, s = sentences.shape
    n = b * s
    v = table.shape[0]
    packed = _pack_fn(VOCAB_PAD)(table.T)
    table_rm = packed.reshape(VOCAB_PAD, EMBED)
    idx = sentences.T.reshape(n).astype(jnp.int32)
    out = _gather_fn(n, VOCAB_PAD)(idx, table_rm)
    return out.reshape(s, b, EMBED).transpose(1, 0, 2)


# speedup vs baseline: 2.2781x; 1.0166x over previous
import functools

import jax
import jax.numpy as jnp
from jax import lax
from jax.experimental import pallas as pl
from jax.experimental.pallas import tpu as pltpu
from jax.experimental.pallas import tpu_sc as plsc

EMBED = 64
CHUNK = 512
TBLOCK = 32768
HALF_SHIFT = (TBLOCK // 2).bit_length() - 1
VOCAB_PAD = 1015808


def _pack_block(tt_ref, p_ref):
    a = tt_ref[...]
    r = lax.broadcasted_iota(jnp.int32, (EMBED, EMBED), 0)
    c = lax.broadcasted_iota(jnp.int32, (EMBED, EMBED), 1)
    ident = (r == c).astype(jnp.float32)
    y = lax.dot_general(
        a, ident, (((0,), (0,)), ((), ())),
        preferred_element_type=jnp.float32)
    half = TBLOCK // 2
    p_ref[...] = jnp.concatenate([y[:half], y[half:]], axis=1)


@functools.cache
def _pack_fn(v_pad: int):
    n_blocks = v_pad // TBLOCK
    assert n_blocks * TBLOCK == v_pad
    return pl.pallas_call(
        _pack_block,
        grid=(n_blocks,),
        in_specs=[pl.BlockSpec((EMBED, TBLOCK), lambda i: (0, i))],
        out_specs=pl.BlockSpec((TBLOCK // 2, 2 * EMBED), lambda i: (i, 0)),
        out_shape=jax.ShapeDtypeStruct((v_pad // 2, 2 * EMBED), jnp.float32),
    )


@functools.cache
def _gather_fn(n_total: int, v_pad: int):
    info = plsc.get_sparse_core_info()
    nc, ns = info.num_cores, info.num_subcores
    nw = nc * ns
    per_w = n_total // nw
    n_chunks = per_w // CHUNK
    assert per_w * nw == n_total and n_chunks * CHUNK == per_w
    assert n_chunks % 2 == 0 and n_chunks >= 6

    mesh = plsc.VectorSubcoreMesh(core_axis_name="c", subcore_axis_name="s")

    @functools.partial(
        pl.kernel,
        mesh=mesh,
        out_type=jax.ShapeDtypeStruct((n_total, EMBED), jnp.float32),
        scratch_types=[
            pltpu.VMEM((CHUNK,), jnp.int32),
            pltpu.VMEM((CHUNK,), jnp.int32),
            pltpu.VMEM((CHUNK, EMBED), jnp.float32),
            pltpu.VMEM((CHUNK, EMBED), jnp.float32),
            pltpu.SemaphoreType.DMA,
            pltpu.SemaphoreType.DMA,
            pltpu.SemaphoreType.DMA,
            pltpu.SemaphoreType.DMA,
            pltpu.SemaphoreType.DMA,
            pltpu.SemaphoreType.DMA,
        ],
        compiler_params=pltpu.CompilerParams(use_tc_tiling_on_sc=False),
    )
    def k(idx_hbm, table_hbm, out_hbm, idx0, idx1, rows0, rows1,
          si0, si1, sg0, sg1, so0, so1):
        idx_v = (idx0, idx1)
        rows_v = (rows0, rows1)
        si = (si0, si1)
        sg = (sg0, sg1)
        so = (so0, so1)
        wid = lax.axis_index("s") * nc + lax.axis_index("c")
        w_base = wid * per_w

        def start_idx(i, b):
            pltpu.async_copy(
                idx_hbm.at[pl.ds(w_base + i * CHUNK, CHUNK)], idx_v[b], si[b])

        def wait_idx(b):
            pltpu.make_async_copy(
                idx_hbm.at[pl.ds(0, CHUNK)], idx_v[b], si[b]).wait()

        def start_out(i, b):
            pltpu.async_copy(
                rows_v[b], out_hbm.at[pl.ds(w_base + i * CHUNK, CHUNK)], so[b])

        def wait_out(b):
            pltpu.make_async_copy(
                rows_v[b], out_hbm.at[pl.ds(0, CHUNK)], so[b]).wait()

        def wait_gather(b):
            pltpu.make_async_copy(
                table_hbm.at[idx_v[b]], rows_v[b], sg[b]).wait()

        def xform_idx(b):
            def tb(kk, carry2):
                t = idx_v[b][pl.ds(kk * 16, 16)]
                u = (t & ~(TBLOCK - 1)) | ((t & (TBLOCK // 2 - 1)) << 1) | ((t >> HALF_SHIFT) & 1)
                idx_v[b][pl.ds(kk * 16, 16)] = u
                return carry2

            lax.fori_loop(0, CHUNK // 16, tb, 0)

        def step(i, b, *, slot_wait, drain_prev, prefetch):
            if slot_wait:
                wait_out(b)
            wait_idx(b)
            xform_idx(b)
            pltpu.async_copy(table_hbm.at[idx_v[b]], rows_v[b], sg[b])
            if drain_prev:
                wait_gather(1 - b)
                start_out(i - 1, 1 - b)
            if prefetch:
                start_idx(i + 1, 1 - b)

        start_idx(0, 0)
        step(0, 0, slot_wait=False, drain_prev=False, prefetch=True)
        step(1, 1, slot_wait=False, drain_prev=True, prefetch=True)

        def outer(j, carry):
            i2 = j * 2
            step(i2, 0, slot_wait=True, drain_prev=True, prefetch=True)
            step(i2 + 1, 1, slot_wait=True, drain_prev=True, prefetch=True)
            return carry

        lax.fori_loop(1, n_chunks // 2 - 1, outer, 0)

        step(n_chunks - 2, 0, slot_wait=True, drain_prev=True, prefetch=True)
        step(n_chunks - 1, 1, slot_wait=True, drain_prev=True, prefetch=False)
        wait_gather(1)
        start_out(n_chunks - 1, 1)
        wait_out(0)
        wait_out(1)

    return k


def kernel(sentences, table):
    b, s = sentences.shape
    n = b * s
    v = table.shape[0]
    packed = _pack_fn(VOCAB_PAD)(table.T)
    table_rm = packed.reshape(VOCAB_PAD, EMBED)
    idx = sentences.T.reshape(n).astype(jnp.int32)
    out = _gather_fn(n, VOCAB_PAD)(idx, table_rm)
    return out.reshape(s, b, EMBED).transpose(1, 0, 2)


# speedup vs baseline: 2.2823x; 1.0018x over previous
import functools

import jax
import jax.numpy as jnp
from jax import lax
from jax.experimental import pallas as pl
from jax.experimental.pallas import tpu as pltpu
from jax.experimental.pallas import tpu_sc as plsc

EMBED = 64
CHUNK = 512
TBLOCK = 32768
HALF_SHIFT = (TBLOCK // 2).bit_length() - 1
VOCAB_PAD = 1015808


def _pack_block(tt_ref, p_ref):
    a = tt_ref[...]
    r = lax.broadcasted_iota(jnp.int32, (EMBED, EMBED), 0)
    c = lax.broadcasted_iota(jnp.int32, (EMBED, EMBED), 1)
    ident = (r == c).astype(jnp.float32)
    y = lax.dot_general(
        a, ident, (((0,), (0,)), ((), ())),
        preferred_element_type=jnp.float32)
    half = TBLOCK // 2
    p_ref[...] = jnp.concatenate([y[:half], y[half:]], axis=1)


@functools.cache
def _pack_fn(v_pad: int):
    n_blocks = v_pad // TBLOCK
    assert n_blocks * TBLOCK == v_pad
    return pl.pallas_call(
        _pack_block,
        grid=(n_blocks,),
        in_specs=[pl.BlockSpec((EMBED, TBLOCK), lambda i: (0, i))],
        out_specs=pl.BlockSpec((TBLOCK // 2, 2 * EMBED), lambda i: (i, 0)),
        out_shape=jax.ShapeDtypeStruct((v_pad // 2, 2 * EMBED), jnp.float32),
    )


@functools.cache
def _gather_fn(n_total: int, v_pad: int):
    info = plsc.get_sparse_core_info()
    nc, ns = info.num_cores, info.num_subcores
    nw = nc * ns
    per_w = n_total // nw
    n_chunks = per_w // CHUNK
    assert per_w * nw == n_total and n_chunks * CHUNK == per_w
    assert n_chunks % 2 == 0 and n_chunks >= 6

    mesh = plsc.VectorSubcoreMesh(core_axis_name="c", subcore_axis_name="s")

    @functools.partial(
        pl.kernel,
        mesh=mesh,
        out_type=jax.ShapeDtypeStruct((n_total, EMBED), jnp.float32),
        scratch_types=[
            pltpu.VMEM((CHUNK,), jnp.int32),
            pltpu.VMEM((CHUNK,), jnp.int32),
            pltpu.VMEM((CHUNK, EMBED), jnp.float32),
            pltpu.VMEM((CHUNK, EMBED), jnp.float32),
            pltpu.SemaphoreType.DMA,
            pltpu.SemaphoreType.DMA,
            pltpu.SemaphoreType.DMA,
            pltpu.SemaphoreType.DMA,
            pltpu.SemaphoreType.DMA,
            pltpu.SemaphoreType.DMA,
        ],
        compiler_params=pltpu.CompilerParams(use_tc_tiling_on_sc=False),
    )
    def k(idx_hbm, table_hbm, out_hbm, idx0, idx1, rows0, rows1,
          si0, si1, sg0, sg1, so0, so1):
        idx_v = (idx0, idx1)
        rows_v = (rows0, rows1)
        si = (si0, si1)
        sg = (sg0, sg1)
        so = (so0, so1)
        wid = lax.axis_index("s") * nc + lax.axis_index("c")
        w_base = wid * per_w

        def start_idx(i, b):
            pltpu.async_copy(
                idx_hbm.at[pl.ds(w_base + i * CHUNK, CHUNK)], idx_v[b], si[b])

        def wait_idx(b):
            pltpu.make_async_copy(
                idx_hbm.at[pl.ds(0, CHUNK)], idx_v[b], si[b]).wait()

        def start_out(i, b):
            pltpu.async_copy(
                rows_v[b], out_hbm.at[pl.ds(w_base + i * CHUNK, CHUNK)], so[b])

        def wait_out(b):
            pltpu.make_async_copy(
                rows_v[b], out_hbm.at[pl.ds(0, CHUNK)], so[b]).wait()

        def wait_gather(b):
            pltpu.make_async_copy(
                table_hbm.at[idx_v[b]], rows_v[b], sg[b]).wait()

        def xform_idx(b):
            def tb(kk, carry2):
                t = idx_v[b][pl.ds(kk * 16, 16)]
                u = (t & ~(TBLOCK - 1)) | ((t & (TBLOCK // 2 - 1)) << 1) | ((t >> HALF_SHIFT) & 1)
                idx_v[b][pl.ds(kk * 16, 16)] = u
                return carry2

            lax.fori_loop(0, CHUNK // 16, tb, 0)

        def step(i, b, *, slot_wait, drain_prev, prefetch):
            if slot_wait:
                wait_out(b)
            wait_idx(b)
            xform_idx(b)
            pltpu.async_copy(table_hbm.at[idx_v[b]], rows_v[b], sg[b])
            if drain_prev:
                wait_gather(1 - b)
                start_out(i - 1, 1 - b)
            if prefetch:
                start_idx(i + 1, 1 - b)

        start_idx(0, 0)
        step(0, 0, slot_wait=False, drain_prev=False, prefetch=True)
        step(1, 1, slot_wait=False, drain_prev=True, prefetch=True)

        def outer(j, carry):
            i2 = j * 2
            step(i2, 0, slot_wait=True, drain_prev=True, prefetch=True)
            step(i2 + 1, 1, slot_wait=True, drain_prev=True, prefetch=True)
            return carry

        lax.fori_loop(1, n_chunks // 2 - 1, outer, 0)

        step(n_chunks - 2, 0, slot_wait=True, drain_prev=True, prefetch=True)
        step(n_chunks - 1, 1, slot_wait=True, drain_prev=True, prefetch=False)
        wait_gather(1)
        start_out(n_chunks - 1, 1)
        wait_out(0)
        wait_out(1)

    return k


def kernel(sentences, table):
    b, s = sentences.shape
    n = b * s
    packed = _pack_fn(VOCAB_PAD)(table.T)
    table_rm = packed.reshape(VOCAB_PAD, EMBED)
    idx = sentences.T.reshape(n).astype(jnp.int32)
    out = _gather_fn(n, VOCAB_PAD)(idx, table_rm)
    return out.reshape(s, b, EMBED).transpose(1, 0, 2)
